# baseline (speedup 1.0000x reference)
import jax
import jax.numpy as jnp
from jax import lax
from jax.experimental import pallas as pl
from jax.experimental.pallas import tpu as pltpu

SEQ = 64
EMBED = 64
NJ = 21
KJ = 32
G = 8
NG = SEQ // G
BB = 512


def _body(pT_ref, cT_ref, pos_ref, pieceT_ref, colorT_ref, out_ref, jt_ref):
    @pl.when(pl.program_id(0) == 0)
    def _build_tables():
        selp = (lax.broadcasted_iota(jnp.int32, (7, KJ), 0)
                == lax.broadcasted_iota(jnp.int32, (7, KJ), 1) // 3)
        selc = ((lax.broadcasted_iota(jnp.int32, (3, KJ), 0)
                 == lax.broadcasted_iota(jnp.int32, (3, KJ), 1) % 3)
                & (lax.broadcasted_iota(jnp.int32, (3, KJ), 1) < NJ))
        joint_t = (
            jnp.dot(pieceT_ref[...], selp.astype(jnp.float32),
                    preferred_element_type=jnp.float32)
            + jnp.dot(colorT_ref[...], selc.astype(jnp.float32),
                      preferred_element_type=jnp.float32))
        jtr = jnp.broadcast_to(joint_t[None], (G, EMBED, KJ)).reshape(
            G * EMBED, KJ)
        rexp = (lax.broadcasted_iota(jnp.int32, (KJ, G * KJ), 0)
                == lax.broadcasted_iota(jnp.int32, (KJ, G * KJ), 1) // G)
        a = jnp.dot(jtr, rexp.astype(jnp.float32),
                    preferred_element_type=jnp.float32)
        mask = (lax.broadcasted_iota(jnp.int32, (G * EMBED, G * KJ), 1) % G
                == lax.broadcasted_iota(jnp.int32, (G * EMBED, G * KJ), 0)
                // EMBED)
        esel = (lax.broadcasted_iota(jnp.int32, (G * EMBED, G), 0) // EMBED
                == lax.broadcasted_iota(jnp.int32, (G * EMBED, G), 1)
                ).astype(jnp.float32)
        dmask = (lax.broadcasted_iota(jnp.int32, (G * EMBED, EMBED), 1)
                 == lax.broadcasted_iota(jnp.int32, (G * EMBED, EMBED), 0)
                 % EMBED)
        for g in range(NG):
            p2 = jnp.dot(esel, pos_ref[g * G:(g + 1) * G, :],
                         preferred_element_type=jnp.float32)
            posg = jnp.sum(jnp.where(dmask, p2, 0.0), axis=1, keepdims=True)
            jt_ref[g] = jnp.where(mask, a + posg, 0.0)

    j2 = pT_ref[...] * 3 + cT_ref[...]
    jsel = lax.broadcasted_iota(jnp.int32, (G * KJ, BB), 0) // G
    for g in range(NG):
        j2g = j2[g * G:(g + 1) * G, :]
        oh = (jnp.broadcast_to(j2g[None], (KJ, G, BB)).reshape(G * KJ, BB)
              == jsel).astype(jnp.float32)
        m = jnp.dot(jt_ref[g], oh, preferred_element_type=jnp.float32)
        out_ref[g * G:(g + 1) * G] = m.reshape(G, EMBED, BB)


def kernel(pieces_ids, color_ids, position_emb, piece_emb, color_emb):
    B = pieces_ids.shape[0]
    pT = pieces_ids.astype(jnp.int32).T
    cT = color_ids.astype(jnp.int32).T
    out_t = pl.pallas_call(
        _body,
        grid=(B // BB,),
        in_specs=[
            pl.BlockSpec((SEQ, BB), lambda i: (0, i)),
            pl.BlockSpec((SEQ, BB), lambda i: (0, i)),
            pl.BlockSpec((SEQ, EMBED), lambda i: (0, 0)),
            pl.BlockSpec((EMBED, 7), lambda i: (0, 0)),
            pl.BlockSpec((EMBED, 3), lambda i: (0, 0)),
        ],
        out_specs=pl.BlockSpec((SEQ, EMBED, BB), lambda i: (0, 0, i)),
        out_shape=jax.ShapeDtypeStruct((SEQ, EMBED, B), jnp.float32),
        scratch_shapes=[
            pltpu.VMEM((NG, G * EMBED, G * KJ), jnp.float32),
        ],
    )(pT, cT, position_emb, piece_emb.T, color_emb.T)
    return jnp.transpose(out_t, (2, 0, 1))
